# Initial kernel scaffold; baseline (speedup 1.0000x reference)
import functools

import jax
import jax.numpy as jnp
from jax import lax
from jax.experimental import pallas as pl
from jax.experimental.pallas import tpu as pltpu
from jax.experimental.pallas import tpu_sc as plsc

N = 10000
M = 5000
P = 320000
D = 128

NC = 2
NS = 16
NW = NC * NS

C = 80
PW = P // NW
NCH = PW // C

MP = 5120
NP = 10240
MZ = MP // NS
NZ = NP // NS

_mesh = plsc.VectorSubcoreMesh(core_axis_name="c", subcore_axis_name="s")


def _edge_agg_body(x_hbm, vidx_hbm, eidx_hbm, z128_hbm, z16_hbm, ones_hbm,
                   xe_out, cnt_out,
                   xe_sh, cnt_sh, vidx_v, eidx_v, rows_v, ones_v, sem):
    c = lax.axis_index("c")
    s = lax.axis_index("s")
    w = c * NS + s
    pltpu.sync_copy(vidx_hbm.at[pl.ds(w * NCH, NCH)], vidx_v)
    pltpu.sync_copy(eidx_hbm.at[pl.ds(w * NCH, NCH)], eidx_v)
    pltpu.sync_copy(ones_hbm, ones_v)
    pltpu.sync_copy(z128_hbm, xe_sh.at[pl.ds(s * MZ, MZ)])
    pltpu.sync_copy(z16_hbm, cnt_sh.at[pl.ds(s * MZ, MZ)])
    plsc.subcore_barrier()

    @pl.loop(0, NCH)
    def _(i):
        pltpu.async_copy(x_hbm.at[vidx_v.at[i]], rows_v, sem).wait()
        pltpu.sync_copy(rows_v, xe_sh.at[eidx_v.at[i]], add=True)
        pltpu.sync_copy(ones_v, cnt_sh.at[eidx_v.at[i]], add=True)

    plsc.subcore_barrier()
    pltpu.sync_copy(xe_sh.at[pl.ds(s * MZ, MZ)], xe_out.at[c, pl.ds(s * MZ, MZ)])
    pltpu.sync_copy(cnt_sh.at[pl.ds(s * MZ, MZ)], cnt_out.at[c, pl.ds(s * MZ, MZ)])


_edge_agg = pl.kernel(
    _edge_agg_body,
    out_type=[
        jax.ShapeDtypeStruct((NC, MP, D), jnp.float32),
        jax.ShapeDtypeStruct((NC, MP, 16), jnp.float32),
    ],
    mesh=_mesh,
    scratch_types=[
        pltpu.VMEM_SHARED((MP, D), jnp.float32),
        pltpu.VMEM_SHARED((MP, 16), jnp.float32),
        pltpu.VMEM((NCH, C), jnp.int32),
        pltpu.VMEM((NCH, C), jnp.int32),
        pltpu.VMEM((C, D), jnp.float32),
        pltpu.VMEM((C, 16), jnp.float32),
        pltpu.SemaphoreType.DMA,
    ],
)


def _vertex_agg_body(xe_hbm, eidx_hbm, vidx_hbm, z128_hbm,
                     xv_out,
                     xv_sh, eidx_v, vidx_v, rows_v, sem):
    c = lax.axis_index("c")
    s = lax.axis_index("s")
    w = c * NS + s
    pltpu.sync_copy(eidx_hbm.at[pl.ds(w * NCH, NCH)], eidx_v)
    pltpu.sync_copy(vidx_hbm.at[pl.ds(w * NCH, NCH)], vidx_v)
    pltpu.sync_copy(z128_hbm, xv_sh.at[pl.ds(s * NZ, NZ)])
    plsc.subcore_barrier()

    @pl.loop(0, NCH)
    def _(i):
        pltpu.async_copy(xe_hbm.at[eidx_v.at[i]], rows_v, sem).wait()
        pltpu.sync_copy(rows_v, xv_sh.at[vidx_v.at[i]], add=True)

    plsc.subcore_barrier()
    pltpu.sync_copy(xv_sh.at[pl.ds(s * NZ, NZ)], xv_out.at[c, pl.ds(s * NZ, NZ)])


_vertex_agg = pl.kernel(
    _vertex_agg_body,
    out_type=jax.ShapeDtypeStruct((NC, NP, D), jnp.float32),
    mesh=_mesh,
    scratch_types=[
        pltpu.VMEM_SHARED((NP, D), jnp.float32),
        pltpu.VMEM((NCH, C), jnp.int32),
        pltpu.VMEM((NCH, C), jnp.int32),
        pltpu.VMEM((C, D), jnp.float32),
        pltpu.SemaphoreType.DMA,
    ],
)


_BR1 = 256
_BR2 = 400


def _norm_body(xe_ref, cnt_ref, dege_ref, out_ref):
    xe = xe_ref[0] + xe_ref[1]
    cnt = cnt_ref[0, :, :1] + cnt_ref[1, :, :1]
    out_ref[...] = xe * (dege_ref[...] / jnp.maximum(cnt, 1.0))


def _edge_norm(xe_parts, cnt_parts, dege_pad):
    return pl.pallas_call(
        _norm_body,
        grid=(MP // _BR1,),
        in_specs=[
            pl.BlockSpec((NC, _BR1, D), lambda i: (0, i, 0)),
            pl.BlockSpec((NC, _BR1, 16), lambda i: (0, i, 0)),
            pl.BlockSpec((_BR1, 1), lambda i: (i, 0)),
        ],
        out_specs=pl.BlockSpec((_BR1, D), lambda i: (i, 0)),
        out_shape=jax.ShapeDtypeStruct((MP, D), jnp.float32),
    )(xe_parts, cnt_parts, dege_pad)


def _final_body(xv_ref, degv_ref, x0_ref, w_ref, p_ref, out_ref):
    a = p_ref[0, 0]
    b = p_ref[0, 1]
    xv = xv_ref[0] + xv_ref[1]
    xi = (1.0 - a) * xv * degv_ref[...] + a * x0_ref[...]
    out_ref[...] = (1.0 - b) * xi + b * jnp.dot(
        xi, w_ref[...], preferred_element_type=jnp.float32)


def _final(xv_parts, degv, x0, w, params):
    return pl.pallas_call(
        _final_body,
        grid=(N // _BR2,),
        in_specs=[
            pl.BlockSpec((NC, _BR2, D), lambda i: (0, i, 0)),
            pl.BlockSpec((_BR2, 1), lambda i: (i, 0)),
            pl.BlockSpec((_BR2, D), lambda i: (i, 0)),
            pl.BlockSpec((D, D), lambda i: (0, 0)),
            pl.BlockSpec((1, 2), lambda i: (0, 0)),
        ],
        out_specs=pl.BlockSpec((_BR2, D), lambda i: (i, 0)),
        out_shape=jax.ShapeDtypeStruct((N, D), jnp.float32),
    )(xv_parts, degv, x0, w, params)


def kernel(X, X0, vertex_idx, edge_idx, degE, degV, W, alpha, beta):
    vidx2d = vertex_idx.reshape(NW * NCH, C)
    eidx2d = edge_idx.reshape(NW * NCH, C)
    z128e = jnp.zeros((MZ, D), jnp.float32)
    z16 = jnp.zeros((MZ, 16), jnp.float32)
    z128v = jnp.zeros((NZ, D), jnp.float32)
    ones = jnp.ones((C, 16), jnp.float32)

    xe_parts, cnt_parts = _edge_agg(X, vidx2d, eidx2d, z128e, z16, ones)

    dege_pad = jnp.pad(degE.astype(jnp.float32), ((0, MP - M), (0, 0)))
    xe = _edge_norm(xe_parts, cnt_parts, dege_pad)

    xv_parts = _vertex_agg(xe, eidx2d, vidx2d, z128v)

    params = jnp.stack([jnp.float32(alpha), jnp.float32(beta)]).reshape(1, 2)
    return _final(xv_parts[:, :N, :], degV.astype(jnp.float32), X0, W, params)


# speedup vs baseline: 3.8617x; 3.8617x over previous
import dataclasses
import functools

import jax
import jax.numpy as jnp
from jax import lax
from jax.experimental import pallas as pl
from jax.experimental.pallas import tpu as pltpu
from jax.experimental.pallas import tpu_sc as plsc

N = 10000
M = 5000
P = 320000
D = 128

NC = 2
NS = 16
NW = NC * NS

C = 80
PW = P // NW
NCH = PW // C

MP = 5120
NP = 10240
MZ = MP // NS
NZ = NP // NS

@functools.lru_cache(maxsize=None)
def _sc_mesh():
    return plsc.VectorSubcoreMesh(core_axis_name="c", subcore_axis_name="s",
                                  num_cores=NC, num_subcores=NS)


def _edge_agg_body(x_hbm, vidx_hbm, eidx_hbm, z128_hbm,
                   xe_out, cnt_out,
                   xe_sh, vidx_v, eidx_v, rows_v, hist_v, sem):
    c = lax.axis_index("c")
    s = lax.axis_index("s")
    w = c * NS + s
    pltpu.sync_copy(vidx_hbm.at[w], vidx_v)
    pltpu.sync_copy(eidx_hbm.at[w], eidx_v)
    pltpu.sync_copy(z128_hbm, xe_sh.at[pl.ds(s * MZ, MZ)])

    @pl.loop(0, MP // 16)
    def _(k):
        hist_v[0, pl.ds(k * 16, 16)] = jnp.zeros((16,), jnp.float32)

    plsc.subcore_barrier()

    ones16 = jnp.ones((16,), jnp.float32)
    row0 = jnp.zeros((16,), jnp.int32)

    @pl.loop(0, NCH)
    def _(i):
        gath = pltpu.async_copy(x_hbm.at[vidx_v.at[i]], rows_v, sem)
        for j in range(C // 16):
            idx = eidx_v[i, pl.ds(j * 16, 16)]
            plsc.addupdate_scatter(hist_v, [row0, idx], ones16)
        gath.wait()
        pltpu.sync_copy(rows_v, xe_sh.at[eidx_v.at[i]], add=True)

    plsc.subcore_barrier()
    pltpu.sync_copy(xe_sh.at[pl.ds(s * MZ, MZ)], xe_out.at[c, pl.ds(s * MZ, MZ)])
    pltpu.sync_copy(hist_v, cnt_out.at[c, s])


@functools.lru_cache(maxsize=None)
def _edge_agg():
    cp = dataclasses.replace(pltpu.CompilerParams(), needs_layout_passes=False)
    return pl.kernel(
        _edge_agg_body,
        out_type=[
            jax.ShapeDtypeStruct((NC, MP, D), jnp.float32),
            jax.ShapeDtypeStruct((NC, NS, 1, MP), jnp.float32),
        ],
        mesh=_sc_mesh(),
        scratch_types=[
            pltpu.VMEM_SHARED((MP, D), jnp.float32),
            pltpu.VMEM((NCH, C), jnp.int32),
            pltpu.VMEM((NCH, C), jnp.int32),
            pltpu.VMEM((C, D), jnp.float32),
            pltpu.VMEM((1, MP), jnp.float32),
            pltpu.SemaphoreType.DMA,
        ],
        compiler_params=cp,
    )


def _vertex_agg_body(xe_hbm, eidx_hbm, vidx_hbm, z128_hbm,
                     xv_out,
                     xv_sh, eidx_v, vidx_v, rows_v, sem):
    c = lax.axis_index("c")
    s = lax.axis_index("s")
    w = c * NS + s
    pltpu.sync_copy(eidx_hbm.at[w], eidx_v)
    pltpu.sync_copy(vidx_hbm.at[w], vidx_v)
    pltpu.sync_copy(z128_hbm, xv_sh.at[pl.ds(s * NZ, NZ)])
    plsc.subcore_barrier()

    @pl.loop(0, NCH)
    def _(i):
        pltpu.async_copy(xe_hbm.at[eidx_v.at[i]], rows_v, sem).wait()
        pltpu.sync_copy(rows_v, xv_sh.at[vidx_v.at[i]], add=True)

    plsc.subcore_barrier()
    pltpu.sync_copy(xv_sh.at[pl.ds(s * NZ, NZ)], xv_out.at[c, pl.ds(s * NZ, NZ)])


@functools.lru_cache(maxsize=None)
def _vertex_agg():
    return pl.kernel(
        _vertex_agg_body,
        out_type=jax.ShapeDtypeStruct((NC, NP, D), jnp.float32),
        mesh=_sc_mesh(),
        scratch_types=[
            pltpu.VMEM_SHARED((NP, D), jnp.float32),
            pltpu.VMEM((NCH, C), jnp.int32),
            pltpu.VMEM((NCH, C), jnp.int32),
            pltpu.VMEM((C, D), jnp.float32),
            pltpu.SemaphoreType.DMA,
        ],
    )


_BR1 = 256
_BR2 = 400


def _norm_body(xe_ref, cnt_ref, dege_ref, out_ref):
    xe = xe_ref[0] + xe_ref[1]
    cnt = jnp.sum(cnt_ref[...], axis=(0, 1, 2))[:, None]
    out_ref[...] = xe * (dege_ref[...] / jnp.maximum(cnt, 1.0))


def _edge_norm(xe_parts, cnt_parts, dege_pad):
    return pl.pallas_call(
        _norm_body,
        grid=(MP // _BR1,),
        in_specs=[
            pl.BlockSpec((NC, _BR1, D), lambda i: (0, i, 0)),
            pl.BlockSpec((NC, NS, 1, _BR1), lambda i: (0, 0, 0, i)),
            pl.BlockSpec((_BR1, 1), lambda i: (i, 0)),
        ],
        out_specs=pl.BlockSpec((_BR1, D), lambda i: (i, 0)),
        out_shape=jax.ShapeDtypeStruct((MP, D), jnp.float32),
    )(xe_parts, cnt_parts, dege_pad)


def _final_body(xv_ref, degv_ref, x0_ref, w_ref, p_ref, out_ref):
    a = p_ref[0, 0]
    b = p_ref[0, 1]
    xv = xv_ref[0] + xv_ref[1]
    xi = (1.0 - a) * xv * degv_ref[...] + a * x0_ref[...]
    out_ref[...] = (1.0 - b) * xi + b * jnp.dot(
        xi, w_ref[...], preferred_element_type=jnp.float32)


def _final(xv_parts, degv, x0, w, params):
    return pl.pallas_call(
        _final_body,
        grid=(N // _BR2,),
        in_specs=[
            pl.BlockSpec((NC, _BR2, D), lambda i: (0, i, 0)),
            pl.BlockSpec((_BR2, 1), lambda i: (i, 0)),
            pl.BlockSpec((_BR2, D), lambda i: (i, 0)),
            pl.BlockSpec((D, D), lambda i: (0, 0)),
            pl.BlockSpec((1, 2), lambda i: (0, 0)),
        ],
        out_specs=pl.BlockSpec((_BR2, D), lambda i: (i, 0)),
        out_shape=jax.ShapeDtypeStruct((N, D), jnp.float32),
    )(xv_parts, degv, x0, w, params)


def kernel(X, X0, vertex_idx, edge_idx, degE, degV, W, alpha, beta):
    vidx2d = vertex_idx.reshape(NW, NCH, C)
    eidx2d = edge_idx.reshape(NW, NCH, C)
    z128e = jnp.zeros((MZ, D), jnp.float32)
    z128v = jnp.zeros((NZ, D), jnp.float32)

    xe_parts, cnt_parts = _edge_agg()(X, vidx2d, eidx2d, z128e)

    dege_pad = jnp.pad(degE.astype(jnp.float32), ((0, MP - M), (0, 0)))
    xe = _edge_norm(xe_parts, cnt_parts, dege_pad)

    xv_parts = _vertex_agg()(xe, eidx2d, vidx2d, z128v)

    params = jnp.stack([jnp.float32(alpha), jnp.float32(beta)]).reshape(1, 2)
    return _final(xv_parts[:, :N, :], degV.astype(jnp.float32), X0, W, params)


# speedup vs baseline: 5.0101x; 1.2974x over previous
import dataclasses
import functools

import jax
import jax.numpy as jnp
from jax import lax
from jax.experimental import pallas as pl
from jax.experimental.pallas import tpu as pltpu
from jax.experimental.pallas import tpu_sc as plsc

N = 10000
M = 5000
P = 320000
D = 128

NC = 2
NS = 16
NW = NC * NS

C = 40
PW = P // NW
NCH = PW // C
NBUF = 5
KOUT = NCH // NBUF

MP = 5120
NP = 10112
MZ = MP // NS
NZ = NP // NS


@functools.lru_cache(maxsize=None)
def _sc_mesh():
    return plsc.VectorSubcoreMesh(core_axis_name="c", subcore_axis_name="s",
                                  num_cores=NC, num_subcores=NS)


def _pipelined_gather_scatter(src_hbm, gidx_v, sidx_v, acc_sh, rows_v,
                              gs, ss):
    def gslice(i):
        return gidx_v.at[pl.ds(i * C, C)]

    def sslice(i):
        return sidx_v.at[pl.ds(i * C, C)]

    @pl.loop(0, KOUT)
    def _(k):
        for b in range(NBUF):
            i = k * NBUF + b
            pltpu.make_async_copy(
                src_hbm.at[gslice(i)], rows_v.at[b], gs[b]).wait()
            pltpu.async_copy(rows_v.at[b], acc_sh.at[sslice(i)], ss[b],
                             add=True)
            bj = (b + 2) % NBUF
            j = i + 2
            jm = j - NBUF

            @pl.when(jnp.logical_and(j >= NBUF, j < NCH))
            def _():
                pltpu.make_async_copy(
                    rows_v.at[bj], acc_sh.at[sslice(jm)], ss[bj]).wait()
                pltpu.async_copy(src_hbm.at[gslice(j)], rows_v.at[bj],
                                 gs[bj])

    for b in range(NBUF):
        i = NCH - NBUF + b
        pltpu.make_async_copy(rows_v.at[b], acc_sh.at[sslice(i)],
                              ss[b]).wait()


def _prologue_gathers(src_hbm, gidx_v, rows_v, gs):
    for b in range(NBUF):
        pltpu.async_copy(src_hbm.at[gidx_v.at[pl.ds(b * C, C)]],
                         rows_v.at[b], gs[b])


def _edge_agg_body(x_hbm, vflat_hbm, eflat_hbm, z128_hbm,
                   xe_out, cnt_out,
                   xe_sh, vflat_v, eflat_v, rows_v, hist_v,
                   g0, g1, g2, g3, g4, s0, s1, s2, s3, s4):
    gs = (g0, g1, g2, g3, g4)
    ss = (s0, s1, s2, s3, s4)
    c = lax.axis_index("c")
    s = lax.axis_index("s")
    w = c * NS + s
    pltpu.sync_copy(vflat_hbm.at[w], vflat_v)
    pltpu.sync_copy(eflat_hbm.at[w], eflat_v)
    pltpu.sync_copy(z128_hbm.at[pl.ds(0, MZ)], xe_sh.at[pl.ds(s * MZ, MZ)])

    @pl.loop(0, MP // 16)
    def _(k):
        hist_v[pl.ds(k * 16, 16)] = jnp.zeros((16,), jnp.float32)

    plsc.subcore_barrier()

    _prologue_gathers(x_hbm, vflat_v, rows_v, gs)

    ones16 = jnp.ones((16,), jnp.float32)

    @pl.loop(0, PW // 16)
    def _(k):
        idx = eflat_v[pl.ds(k * 16, 16)]
        plsc.addupdate_scatter(hist_v, [idx], ones16)

    _pipelined_gather_scatter(x_hbm, vflat_v, eflat_v, xe_sh, rows_v, gs, ss)

    plsc.subcore_barrier()
    pltpu.sync_copy(xe_sh.at[pl.ds(s * MZ, MZ)], xe_out.at[c, pl.ds(s * MZ, MZ)])
    pltpu.sync_copy(hist_v, cnt_out.at[c, pl.ds(s * MP, MP)])


@functools.lru_cache(maxsize=None)
def _edge_agg():
    cp = dataclasses.replace(pltpu.CompilerParams(), needs_layout_passes=False)
    return pl.kernel(
        _edge_agg_body,
        out_type=[
            jax.ShapeDtypeStruct((NC, MP, D), jnp.float32),
            jax.ShapeDtypeStruct((NC, NS * MP), jnp.float32),
        ],
        mesh=_sc_mesh(),
        scratch_types=[
            pltpu.VMEM_SHARED((MP, D), jnp.float32),
            pltpu.VMEM((PW,), jnp.int32),
            pltpu.VMEM((PW,), jnp.int32),
            pltpu.VMEM((NBUF, C, D), jnp.float32),
            pltpu.VMEM((MP,), jnp.float32),
        ] + [pltpu.SemaphoreType.DMA] * (2 * NBUF),
        compiler_params=cp,
    )


def _vertex_agg_body(xe_hbm, eflat_hbm, vflat_hbm, z128_hbm,
                     xv_out,
                     xv_sh, eflat_v, vflat_v, rows_v,
                     g0, g1, g2, g3, g4, s0, s1, s2, s3, s4):
    gs = (g0, g1, g2, g3, g4)
    ss = (s0, s1, s2, s3, s4)
    c = lax.axis_index("c")
    s = lax.axis_index("s")
    w = c * NS + s
    pltpu.sync_copy(eflat_hbm.at[w], eflat_v)
    pltpu.sync_copy(vflat_hbm.at[w], vflat_v)
    pltpu.sync_copy(z128_hbm.at[pl.ds(0, NZ)], xv_sh.at[pl.ds(s * NZ, NZ)])
    plsc.subcore_barrier()

    _prologue_gathers(xe_hbm, eflat_v, rows_v, gs)
    _pipelined_gather_scatter(xe_hbm, eflat_v, vflat_v, xv_sh, rows_v, gs, ss)

    plsc.subcore_barrier()
    pltpu.sync_copy(xv_sh.at[pl.ds(s * NZ, NZ)], xv_out.at[c, pl.ds(s * NZ, NZ)])


@functools.lru_cache(maxsize=None)
def _vertex_agg():
    return pl.kernel(
        _vertex_agg_body,
        out_type=jax.ShapeDtypeStruct((NC, NP, D), jnp.float32),
        mesh=_sc_mesh(),
        scratch_types=[
            pltpu.VMEM_SHARED((NP, D), jnp.float32),
            pltpu.VMEM((PW,), jnp.int32),
            pltpu.VMEM((PW,), jnp.int32),
            pltpu.VMEM((NBUF, C, D), jnp.float32),
        ] + [pltpu.SemaphoreType.DMA] * (2 * NBUF),
    )


_BR1 = 256
_BR2 = 400


def _norm_body(xe_ref, cnt_ref, dege_ref, out_ref):
    xe = xe_ref[0] + xe_ref[1]
    cnt = jnp.sum(cnt_ref[...], axis=0)[:, None]
    out_ref[...] = xe * (dege_ref[...] / jnp.maximum(cnt, 1.0))


def _edge_norm(xe_parts, cnt_parts, dege_pad):
    return pl.pallas_call(
        _norm_body,
        grid=(MP // _BR1,),
        in_specs=[
            pl.BlockSpec((NC, _BR1, D), lambda i: (0, i, 0)),
            pl.BlockSpec((NC * NS, _BR1), lambda i: (0, i)),
            pl.BlockSpec((_BR1, 1), lambda i: (i, 0)),
        ],
        out_specs=pl.BlockSpec((_BR1, D), lambda i: (i, 0)),
        out_shape=jax.ShapeDtypeStruct((MP, D), jnp.float32),
    )(xe_parts, cnt_parts, dege_pad)


def _final_body(xv_ref, degv_ref, x0_ref, w_ref, p_ref, out_ref):
    a = p_ref[0, 0]
    b = p_ref[0, 1]
    xv = xv_ref[0] + xv_ref[1]
    xi = (1.0 - a) * xv * degv_ref[...] + a * x0_ref[...]
    out_ref[...] = (1.0 - b) * xi + b * jnp.dot(
        xi, w_ref[...], preferred_element_type=jnp.float32)


def _final(xv_parts, degv, x0, w, params):
    return pl.pallas_call(
        _final_body,
        grid=(N // _BR2,),
        in_specs=[
            pl.BlockSpec((NC, _BR2, D), lambda i: (0, i, 0)),
            pl.BlockSpec((_BR2, 1), lambda i: (i, 0)),
            pl.BlockSpec((_BR2, D), lambda i: (i, 0)),
            pl.BlockSpec((D, D), lambda i: (0, 0)),
            pl.BlockSpec((1, 2), lambda i: (0, 0)),
        ],
        out_specs=pl.BlockSpec((_BR2, D), lambda i: (i, 0)),
        out_shape=jax.ShapeDtypeStruct((N, D), jnp.float32),
    )(xv_parts, degv, x0, w, params)


def kernel(X, X0, vertex_idx, edge_idx, degE, degV, W, alpha, beta):
    vflat = vertex_idx.reshape(NW, PW)
    eflat = edge_idx.reshape(NW, PW)
    z128v = jnp.zeros((NZ, D), jnp.float32)

    xe_parts, cnt_parts = _edge_agg()(X, vflat, eflat, z128v)

    dege_pad = jnp.pad(degE.astype(jnp.float32), ((0, MP - M), (0, 0)))
    xe = _edge_norm(xe_parts, cnt_parts.reshape(NC * NS, MP), dege_pad)

    xv_parts = _vertex_agg()(xe, eflat, vflat, z128v)

    params = jnp.stack([jnp.float32(alpha), jnp.float32(beta)]).reshape(1, 2)
    return _final(xv_parts, degV.astype(jnp.float32), X0, W, params)


# speedup vs baseline: 9.8484x; 1.9657x over previous
import dataclasses
import functools

import jax
import jax.numpy as jnp
from jax import lax
from jax.experimental import pallas as pl
from jax.experimental.pallas import tpu as pltpu
from jax.experimental.pallas import tpu_sc as plsc

N = 10000
M = 5000
P = 320000
D = 128

NC = 2
NS = 16
NW = NC * NS

C = 40
PW = P // NW
NCH = PW // C
NBUF = 5
KOUT = NCH // NBUF

MP = 5120
NP = 10112
MZ = MP // NS
NZ = NP // NS


@functools.lru_cache(maxsize=None)
def _sc_mesh():
    return plsc.VectorSubcoreMesh(core_axis_name="c", subcore_axis_name="s",
                                  num_cores=NC, num_subcores=NS)


def _pipelined_gather_scatter(src_hbm, gidx_v, sidx_v, acc_sh, rows_v,
                              gs, ss):
    def gslice(i):
        return gidx_v.at[pl.ds(i * C, C)]

    def sslice(i):
        return sidx_v.at[pl.ds(i * C, C)]

    @pl.loop(0, KOUT)
    def _(k):
        for b in range(NBUF):
            i = k * NBUF + b
            pltpu.make_async_copy(
                src_hbm.at[gslice(i)], rows_v.at[b], gs[b]).wait()
            pltpu.async_copy(rows_v.at[b], acc_sh.at[sslice(i)], ss[b],
                             add=True)
            bj = (b + 2) % NBUF
            j = i + 2
            jm = j - NBUF

            @pl.when(jnp.logical_and(j >= NBUF, j < NCH))
            def _():
                pltpu.make_async_copy(
                    rows_v.at[bj], acc_sh.at[sslice(jm)], ss[bj]).wait()
                pltpu.async_copy(src_hbm.at[gslice(j)], rows_v.at[bj],
                                 gs[bj])

    for b in range(NBUF):
        i = NCH - NBUF + b
        pltpu.make_async_copy(rows_v.at[b], acc_sh.at[sslice(i)],
                              ss[b]).wait()


def _prologue_gathers(src_hbm, gidx_v, rows_v, gs):
    for b in range(NBUF):
        pltpu.async_copy(src_hbm.at[gidx_v.at[pl.ds(b * C, C)]],
                         rows_v.at[b], gs[b])


def _edge_agg_body(x_hbm, vflat_hbm, eflat_hbm, z128_hbm,
                   xe_out, cnt_out,
                   xe_sh, vflat_v, eflat_v, rows_v, hist_v,
                   g0, g1, g2, g3, g4, s0, s1, s2, s3, s4):
    gs = (g0, g1, g2, g3, g4)
    ss = (s0, s1, s2, s3, s4)
    c = lax.axis_index("c")
    s = lax.axis_index("s")
    w = c * NS + s
    pltpu.sync_copy(vflat_hbm.at[w], vflat_v)
    pltpu.sync_copy(eflat_hbm.at[w], eflat_v)
    pltpu.sync_copy(z128_hbm.at[pl.ds(0, MZ)], xe_sh.at[pl.ds(s * MZ, MZ)])

    @pl.loop(0, MP // 16)
    def _(k):
        hist_v[pl.ds(k * 16, 16)] = jnp.zeros((16,), jnp.float32)

    plsc.subcore_barrier()

    _prologue_gathers(x_hbm, vflat_v, rows_v, gs)

    ones16 = jnp.ones((16,), jnp.float32)

    @pl.loop(0, PW // 16)
    def _(k):
        idx = eflat_v[pl.ds(k * 16, 16)]
        plsc.addupdate_scatter(hist_v, [idx], ones16)

    _pipelined_gather_scatter(x_hbm, vflat_v, eflat_v, xe_sh, rows_v, gs, ss)

    plsc.subcore_barrier()
    pltpu.sync_copy(xe_sh.at[pl.ds(s * MZ, MZ)], xe_out.at[c, pl.ds(s * MZ, MZ)])
    pltpu.sync_copy(hist_v, cnt_out.at[c, pl.ds(s * MP, MP)])


@functools.lru_cache(maxsize=None)
def _edge_agg():
    cp = dataclasses.replace(pltpu.CompilerParams(), needs_layout_passes=False)
    return pl.kernel(
        _edge_agg_body,
        out_type=[
            jax.ShapeDtypeStruct((NC, MP, D), jnp.float32),
            jax.ShapeDtypeStruct((NC, NS * MP), jnp.float32),
        ],
        mesh=_sc_mesh(),
        scratch_types=[
            pltpu.VMEM_SHARED((MP, D), jnp.float32),
            pltpu.VMEM((PW,), jnp.int32),
            pltpu.VMEM((PW,), jnp.int32),
            pltpu.VMEM((NBUF, C, D), jnp.float32),
            pltpu.VMEM((MP,), jnp.float32),
        ] + [pltpu.SemaphoreType.DMA] * (2 * NBUF),
        compiler_params=cp,
    )


def _vertex_agg_body(xe_hbm, eflat_hbm, vflat_hbm, z128_hbm,
                     xv_out,
                     xv_sh, eflat_v, vflat_v, rows_v,
                     g0, g1, g2, g3, g4, s0, s1, s2, s3, s4):
    gs = (g0, g1, g2, g3, g4)
    ss = (s0, s1, s2, s3, s4)
    c = lax.axis_index("c")
    s = lax.axis_index("s")
    w = c * NS + s
    pltpu.sync_copy(eflat_hbm.at[w], eflat_v)
    pltpu.sync_copy(vflat_hbm.at[w], vflat_v)
    pltpu.sync_copy(z128_hbm.at[pl.ds(0, NZ)], xv_sh.at[pl.ds(s * NZ, NZ)])
    plsc.subcore_barrier()

    _prologue_gathers(xe_hbm, eflat_v, rows_v, gs)
    _pipelined_gather_scatter(xe_hbm, eflat_v, vflat_v, xv_sh, rows_v, gs, ss)

    plsc.subcore_barrier()
    pltpu.sync_copy(xv_sh.at[pl.ds(s * NZ, NZ)], xv_out.at[c, pl.ds(s * NZ, NZ)])


@functools.lru_cache(maxsize=None)
def _vertex_agg():
    return pl.kernel(
        _vertex_agg_body,
        out_type=jax.ShapeDtypeStruct((NC, NP, D), jnp.float32),
        mesh=_sc_mesh(),
        scratch_types=[
            pltpu.VMEM_SHARED((NP, D), jnp.float32),
            pltpu.VMEM((PW,), jnp.int32),
            pltpu.VMEM((PW,), jnp.int32),
            pltpu.VMEM((NBUF, C, D), jnp.float32),
        ] + [pltpu.SemaphoreType.DMA] * (2 * NBUF),
    )


_BR1 = 256
_BR2 = 400


def _norm_body(xe_ref, cnt_ref, dege_ref, out_ref):
    xe = xe_ref[0] + xe_ref[1]
    cnt = jnp.sum(cnt_ref[...], axis=0)[:, None]
    out_ref[...] = xe * (dege_ref[...] / jnp.maximum(cnt, 1.0))


def _edge_norm(xe_parts, cnt_parts, dege_pad):
    return pl.pallas_call(
        _norm_body,
        grid=(MP // _BR1,),
        in_specs=[
            pl.BlockSpec((NC, _BR1, D), lambda i: (0, i, 0)),
            pl.BlockSpec((NC * NS, _BR1), lambda i: (0, i)),
            pl.BlockSpec((_BR1, 1), lambda i: (i, 0)),
        ],
        out_specs=pl.BlockSpec((_BR1, D), lambda i: (i, 0)),
        out_shape=jax.ShapeDtypeStruct((MP, D), jnp.float32),
    )(xe_parts, cnt_parts, dege_pad)


def _final_body(xv_ref, degv_ref, x0_ref, w_ref, p_ref, out_ref):
    a = p_ref[0, 0]
    b = p_ref[0, 1]
    xv = xv_ref[0] + xv_ref[1]
    xi = (1.0 - a) * xv * degv_ref[...] + a * x0_ref[...]
    out_ref[...] = (1.0 - b) * xi + b * jnp.dot(
        xi, w_ref[...], preferred_element_type=jnp.float32)


def _final(xv_parts, degv, x0, w, params):
    return pl.pallas_call(
        _final_body,
        grid=(N // _BR2,),
        in_specs=[
            pl.BlockSpec((NC, _BR2, D), lambda i: (0, i, 0)),
            pl.BlockSpec((_BR2, 1), lambda i: (i, 0)),
            pl.BlockSpec((_BR2, D), lambda i: (i, 0)),
            pl.BlockSpec((D, D), lambda i: (0, 0)),
            pl.BlockSpec((1, 2), lambda i: (0, 0)),
        ],
        out_specs=pl.BlockSpec((_BR2, D), lambda i: (i, 0)),
        out_shape=jax.ShapeDtypeStruct((N, D), jnp.float32),
    )(xv_parts, degv, x0, w, params)


def kernel(X, X0, vertex_idx, edge_idx, degE, degV, W, alpha, beta):
    vflat = vertex_idx.reshape(NW, PW)
    eflat = edge_idx.reshape(NW, PW)
    z128v = jnp.zeros((NZ, D), jnp.float32)

    xe_parts, cnt_parts = _edge_agg()(X, vflat, eflat, z128v)

    dege_pad = jnp.pad(degE.astype(jnp.float32), ((0, MP - M), (0, 0)))
    xe = _edge_norm(xe_parts, cnt_parts.reshape(NC * NS, MP), dege_pad)

    eperm = edge_idx.reshape(128, P // 128).T.reshape(NW, PW)
    vperm = vertex_idx.reshape(128, P // 128).T.reshape(NW, PW)
    xv_parts = _vertex_agg()(xe, eperm, vperm, z128v)

    params = jnp.stack([jnp.float32(alpha), jnp.float32(beta)]).reshape(1, 2)
    return _final(xv_parts, degV.astype(jnp.float32), X0, W, params)


# speedup vs baseline: 10.6032x; 1.0766x over previous
import dataclasses
import functools

import jax
import jax.numpy as jnp
from jax import lax
from jax.experimental import pallas as pl
from jax.experimental.pallas import tpu as pltpu
from jax.experimental.pallas import tpu_sc as plsc

N = 10000
M = 5000
P = 320000
D = 128

NC = 2
NS = 16
NW = NC * NS

PW = P // NW
CA = 80
CB = 40
NBUF = 5

MP = 5120
NP = 10112
MZ = MP // NS
NZ = NP // NS


@functools.lru_cache(maxsize=None)
def _sc_mesh():
    return plsc.VectorSubcoreMesh(core_axis_name="c", subcore_axis_name="s",
                                  num_cores=NC, num_subcores=NS)


def _pipelined_gather_scatter(src_hbm, gidx_v, sidx_v, acc_sh, rows_v,
                              gs, ss, c_sz):
    nch = PW // c_sz

    def gslice(i):
        return gidx_v.at[pl.ds(i * c_sz, c_sz)]

    def sslice(i):
        return sidx_v.at[pl.ds(i * c_sz, c_sz)]

    @pl.loop(0, nch // NBUF)
    def _(k):
        for b in range(NBUF):
            i = k * NBUF + b
            pltpu.make_async_copy(
                src_hbm.at[gslice(i)], rows_v.at[b], gs[b]).wait()
            pltpu.async_copy(rows_v.at[b], acc_sh.at[sslice(i)], ss[b],
                             add=True)
            bj = (b + 2) % NBUF
            j = i + 2
            jm = j - NBUF

            @pl.when(jnp.logical_and(j >= NBUF, j < nch))
            def _():
                pltpu.make_async_copy(
                    rows_v.at[bj], acc_sh.at[sslice(jm)], ss[bj]).wait()
                pltpu.async_copy(src_hbm.at[gslice(j)], rows_v.at[bj],
                                 gs[bj])

    for b in range(NBUF):
        i = nch - NBUF + b
        pltpu.make_async_copy(rows_v.at[b], acc_sh.at[sslice(i)],
                              ss[b]).wait()


def _prologue_gathers(src_hbm, gidx_v, rows_v, gs, c_sz):
    for b in range(NBUF):
        pltpu.async_copy(src_hbm.at[gidx_v.at[pl.ds(b * c_sz, c_sz)]],
                         rows_v.at[b], gs[b])


def _edge_agg_body(x_hbm, vflat_hbm, eflat_hbm, z128_hbm,
                   xe_out, cnt_out,
                   xe_sh, vflat_v, eflat_v, rows_v, hist_v,
                   g0, g1, g2, g3, g4, s0, s1, s2, s3, s4):
    gs = (g0, g1, g2, g3, g4)
    ss = (s0, s1, s2, s3, s4)
    c = lax.axis_index("c")
    s = lax.axis_index("s")
    w = c * NS + s
    pltpu.sync_copy(vflat_hbm.at[w], vflat_v)
    pltpu.sync_copy(eflat_hbm.at[w], eflat_v)
    pltpu.sync_copy(z128_hbm.at[pl.ds(0, MZ)], xe_sh.at[pl.ds(s * MZ, MZ)])

    @pl.loop(0, MP // 16)
    def _(k):
        hist_v[pl.ds(k * 16, 16)] = jnp.zeros((16,), jnp.float32)

    plsc.subcore_barrier()

    _prologue_gathers(x_hbm, vflat_v, rows_v, gs, CA)

    ones16 = jnp.ones((16,), jnp.float32)

    @pl.loop(0, PW // 16)
    def _(k):
        idx = eflat_v[pl.ds(k * 16, 16)]
        plsc.addupdate_scatter(hist_v, [idx], ones16)

    _pipelined_gather_scatter(x_hbm, vflat_v, eflat_v, xe_sh, rows_v, gs, ss,
                              CA)

    plsc.subcore_barrier()
    pltpu.sync_copy(xe_sh.at[pl.ds(s * MZ, MZ)], xe_out.at[c, pl.ds(s * MZ, MZ)])
    pltpu.sync_copy(hist_v, cnt_out.at[c, pl.ds(s * MP, MP)])


@functools.lru_cache(maxsize=None)
def _edge_agg():
    cp = dataclasses.replace(pltpu.CompilerParams(), needs_layout_passes=False)
    return pl.kernel(
        _edge_agg_body,
        out_type=[
            jax.ShapeDtypeStruct((NC, MP, D), jnp.float32),
            jax.ShapeDtypeStruct((NC, NS * MP), jnp.float32),
        ],
        mesh=_sc_mesh(),
        scratch_types=[
            pltpu.VMEM_SHARED((MP, D), jnp.float32),
            pltpu.VMEM((PW,), jnp.int32),
            pltpu.VMEM((PW,), jnp.int32),
            pltpu.VMEM((NBUF, CA, D), jnp.float32),
            pltpu.VMEM((MP,), jnp.float32),
        ] + [pltpu.SemaphoreType.DMA] * (2 * NBUF),
        compiler_params=cp,
    )


def _vertex_agg_body(xe_hbm, eflat_hbm, vflat_hbm, z128_hbm,
                     xv_out,
                     xv_sh, eflat_v, vflat_v, rows_v,
                     g0, g1, g2, g3, g4, s0, s1, s2, s3, s4):
    gs = (g0, g1, g2, g3, g4)
    ss = (s0, s1, s2, s3, s4)
    c = lax.axis_index("c")
    s = lax.axis_index("s")
    w = c * NS + s
    pltpu.sync_copy(eflat_hbm.at[w], eflat_v)
    pltpu.sync_copy(vflat_hbm.at[w], vflat_v)
    pltpu.sync_copy(z128_hbm.at[pl.ds(0, NZ)], xv_sh.at[pl.ds(s * NZ, NZ)])
    plsc.subcore_barrier()

    _prologue_gathers(xe_hbm, eflat_v, rows_v, gs, CB)
    _pipelined_gather_scatter(xe_hbm, eflat_v, vflat_v, xv_sh, rows_v, gs, ss,
                              CB)

    plsc.subcore_barrier()
    pltpu.sync_copy(xv_sh.at[pl.ds(s * NZ, NZ)], xv_out.at[c, pl.ds(s * NZ, NZ)])


@functools.lru_cache(maxsize=None)
def _vertex_agg():
    return pl.kernel(
        _vertex_agg_body,
        out_type=jax.ShapeDtypeStruct((NC, NP, D), jnp.float32),
        mesh=_sc_mesh(),
        scratch_types=[
            pltpu.VMEM_SHARED((NP, D), jnp.float32),
            pltpu.VMEM((PW,), jnp.int32),
            pltpu.VMEM((PW,), jnp.int32),
            pltpu.VMEM((NBUF, CB, D), jnp.float32),
        ] + [pltpu.SemaphoreType.DMA] * (2 * NBUF),
    )


_BR1 = 256
_BR2 = 400


def _norm_body(xe_ref, cnt_ref, dege_ref, out_ref):
    xe = xe_ref[0] + xe_ref[1]
    cnt = jnp.sum(cnt_ref[...], axis=0)[:, None]
    out_ref[...] = xe * (dege_ref[...] / jnp.maximum(cnt, 1.0))


def _edge_norm(xe_parts, cnt_parts, dege_pad):
    return pl.pallas_call(
        _norm_body,
        grid=(MP // _BR1,),
        in_specs=[
            pl.BlockSpec((NC, _BR1, D), lambda i: (0, i, 0)),
            pl.BlockSpec((NC * NS, _BR1), lambda i: (0, i)),
            pl.BlockSpec((_BR1, 1), lambda i: (i, 0)),
        ],
        out_specs=pl.BlockSpec((_BR1, D), lambda i: (i, 0)),
        out_shape=jax.ShapeDtypeStruct((MP, D), jnp.float32),
    )(xe_parts, cnt_parts, dege_pad)


def _final_body(xv_ref, degv_ref, x0_ref, w_ref, p_ref, out_ref):
    a = p_ref[0, 0]
    b = p_ref[0, 1]
    xv = xv_ref[0] + xv_ref[1]
    xi = (1.0 - a) * xv * degv_ref[...] + a * x0_ref[...]
    out_ref[...] = (1.0 - b) * xi + b * jnp.dot(
        xi, w_ref[...], preferred_element_type=jnp.float32)


def _final(xv_parts, degv, x0, w, params):
    return pl.pallas_call(
        _final_body,
        grid=(N // _BR2,),
        in_specs=[
            pl.BlockSpec((NC, _BR2, D), lambda i: (0, i, 0)),
            pl.BlockSpec((_BR2, 1), lambda i: (i, 0)),
            pl.BlockSpec((_BR2, D), lambda i: (i, 0)),
            pl.BlockSpec((D, D), lambda i: (0, 0)),
            pl.BlockSpec((1, 2), lambda i: (0, 0)),
        ],
        out_specs=pl.BlockSpec((_BR2, D), lambda i: (i, 0)),
        out_shape=jax.ShapeDtypeStruct((N, D), jnp.float32),
    )(xv_parts, degv, x0, w, params)


def kernel(X, X0, vertex_idx, edge_idx, degE, degV, W, alpha, beta):
    vflat = vertex_idx.reshape(NW, PW)
    eflat = edge_idx.reshape(NW, PW)
    z128v = jnp.zeros((NZ, D), jnp.float32)

    xe_parts, cnt_parts = _edge_agg()(X, vflat, eflat, z128v)

    dege_pad = jnp.pad(degE.astype(jnp.float32), ((0, MP - M), (0, 0)))
    xe = _edge_norm(xe_parts, cnt_parts.reshape(NC * NS, MP), dege_pad)

    eperm = edge_idx.reshape(128, P // 128).T.reshape(NW, PW)
    vperm = vertex_idx.reshape(128, P // 128).T.reshape(NW, PW)
    xv_parts = _vertex_agg()(xe, eperm, vperm, z128v)

    params = jnp.stack([jnp.float32(alpha), jnp.float32(beta)]).reshape(1, 2)
    return _final(xv_parts, degV.astype(jnp.float32), X0, W, params)
